# Initial kernel scaffold; baseline (speedup 1.0000x reference)
import jax
import jax.numpy as jnp
from jax.experimental import pallas as pl


def kernel(inputs):
    raise NotImplementedError("write your pallas kernel here")


# speedup vs baseline: 14.9986x; 14.9986x over previous
import jax
import jax.numpy as jnp
from jax.experimental import pallas as pl

_ROWS_PER_BLOCK = 8
_N_ITERS = 24


def _sparsemax_rows(x_ref, o_ref):
    x = x_ref[...]
    m = jnp.max(x, axis=1, keepdims=True)

    def body(_, carry):
        lo, hi = carry
        t = 0.5 * (lo + hi)
        s = jnp.sum(jnp.maximum(x - t, 0.0), axis=1, keepdims=True)
        ge = s >= 1.0
        return jnp.where(ge, t, lo), jnp.where(ge, hi, t)

    lo, _ = jax.lax.fori_loop(0, _N_ITERS, body, (m - 1.0, m))
    above = x > lo
    k = jnp.sum(above.astype(jnp.float32), axis=1, keepdims=True)
    s = jnp.sum(jnp.where(above, x, 0.0), axis=1, keepdims=True)
    tau = (s - 1.0) / k
    o_ref[...] = jnp.maximum(x - tau, 0.0)


def kernel(inputs):
    b, n = inputs.shape
    return pl.pallas_call(
        _sparsemax_rows,
        grid=(b // _ROWS_PER_BLOCK,),
        in_specs=[pl.BlockSpec((_ROWS_PER_BLOCK, n), lambda i: (i, 0))],
        out_specs=pl.BlockSpec((_ROWS_PER_BLOCK, n), lambda i: (i, 0)),
        out_shape=jax.ShapeDtypeStruct((b, n), inputs.dtype),
    )(inputs)


# speedup vs baseline: 21.4448x; 1.4298x over previous
import jax
import jax.numpy as jnp
from jax.experimental import pallas as pl

_ROWS_PER_BLOCK = 8
_N_ITERS = 16


def _sparsemax_rows(x_ref, o_ref):
    x = x_ref[...]
    m = jnp.max(x, axis=1, keepdims=True)

    def body(_, carry):
        lo, hi = carry
        t = 0.5 * (lo + hi)
        s = jnp.sum(jnp.maximum(x - t, 0.0), axis=1, keepdims=True)
        ge = s >= 1.0
        return jnp.where(ge, t, lo), jnp.where(ge, hi, t)

    lo, _ = jax.lax.fori_loop(0, _N_ITERS, body, (m - 1.0, m))
    above = x > lo
    k = jnp.sum(above.astype(jnp.float32), axis=1, keepdims=True)
    s = jnp.sum(jnp.where(above, x, 0.0), axis=1, keepdims=True)
    tau = (s - 1.0) / k
    o_ref[...] = jnp.maximum(x - tau, 0.0)


def kernel(inputs):
    b, n = inputs.shape
    return pl.pallas_call(
        _sparsemax_rows,
        grid=(b // _ROWS_PER_BLOCK,),
        in_specs=[pl.BlockSpec((_ROWS_PER_BLOCK, n), lambda i: (i, 0))],
        out_specs=pl.BlockSpec((_ROWS_PER_BLOCK, n), lambda i: (i, 0)),
        out_shape=jax.ShapeDtypeStruct((b, n), inputs.dtype),
    )(inputs)
